# baseline (speedup 1.0000x reference)
import functools

import jax
import jax.numpy as jnp
from jax import lax
from jax.experimental import pallas as pl
from jax.experimental.pallas import tpu as pltpu
from jax.experimental.pallas import tpu_sc as plsc

B = 16384
NF = 8
D = 64
HID = NF * D

NW = 32
CHUNK = 128

@functools.cache
def _mesh():
    return plsc.VectorSubcoreMesh(core_axis_name="c", subcore_axis_name="s")


def _wid():
    return lax.axis_index("s") * 2 + lax.axis_index("c")


ROWS_A = B // CHUNK // NW


@functools.cache
def _fields_gather():
    @functools.partial(
        pl.kernel,
        mesh=_mesh(),
        out_type=jax.ShapeDtypeStruct((B, NF), jnp.int32),
        scratch_types=[
            pltpu.VMEM((ROWS_A, CHUNK), jnp.int32),
            pltpu.VMEM((CHUNK, NF), jnp.int32),
            pltpu.SemaphoreType.DMA,
        ],
        compiler_params=pltpu.CompilerParams(use_tc_tiling_on_sc=False),
    )
    def body(ids_hbm, table_hbm, out_hbm, idx_v, rows_v, sem):
        w = _wid()
        pltpu.sync_copy(ids_hbm.at[pl.ds(w * ROWS_A, ROWS_A)], idx_v)
        for j in range(ROWS_A):
            pltpu.async_copy(table_hbm.at[idx_v.at[j]], rows_v, sem).wait()
            pltpu.sync_copy(rows_v, out_hbm.at[pl.ds(w * ROWS_A * CHUNK + j * CHUNK, CHUNK)])

    return body


N_IDS = 2 * B * NF
ROWS_B = N_IDS // CHUNK // NW


@functools.cache
def _embed_gather():
    @functools.partial(
        pl.kernel,
        mesh=_mesh(),
        out_type=[
            jax.ShapeDtypeStruct((B * NF, D), jnp.float32),
            jax.ShapeDtypeStruct((B * NF, D), jnp.float32),
        ],
        scratch_types=[
            pltpu.VMEM((ROWS_B, CHUNK), jnp.int32),
            pltpu.VMEM((CHUNK, D), jnp.float32),
            pltpu.SemaphoreType.DMA,
        ],
        compiler_params=pltpu.CompilerParams(use_tc_tiling_on_sc=False),
    )
    def body(ids_hbm, table_hbm, out_u, out_i, idx_v, buf_v, sem):
        w = _wid()
        pltpu.sync_copy(ids_hbm.at[pl.ds(w * ROWS_B, ROWS_B)], idx_v)
        is_user = w < (NW // 2)
        base = jnp.where(is_user, w, w - NW // 2) * (ROWS_B * CHUNK)

        def step(j, carry):
            pltpu.async_copy(table_hbm.at[idx_v.at[j]], buf_v, sem).wait()

            @pl.when(is_user)
            def _():
                pltpu.sync_copy(buf_v, out_u.at[pl.ds(base + j * CHUNK, CHUNK)])

            @pl.when(jnp.logical_not(is_user))
            def _():
                pltpu.sync_copy(buf_v, out_i.at[pl.ds(base + j * CHUNK, CHUNK)])

            return carry

        lax.fori_loop(0, ROWS_B, step, 0)

    return body


BLK = 1024
NB = B // BLK


def _mlp_body(ue, ie, uW1, ub1, uW2, ub2, uW3, ub3, iW1, ib1, iW2, ib2, out):
    f32 = jnp.float32
    h = jnp.maximum(jnp.dot(ue[...], uW1[...], preferred_element_type=f32) + ub1[...], 0.0)
    h = jnp.maximum(jnp.dot(h, uW2[...], preferred_element_type=f32) + ub2[...], 0.0)
    uv = jnp.dot(h, uW3[...], preferred_element_type=f32) + ub3[...]
    g = jnp.maximum(jnp.dot(ie[...], iW1[...], preferred_element_type=f32) + ib1[...], 0.0)
    iv = jnp.dot(g, iW2[...], preferred_element_type=f32) + ib2[...]
    out[...] = jnp.sum(uv * iv, axis=1)


def _full(shape):
    return pl.BlockSpec(shape, lambda i: (0, 0))


_mlp_call = pl.pallas_call(
    _mlp_body,
    grid=(NB,),
    in_specs=[
        pl.BlockSpec((BLK, HID), lambda i: (i, 0)),
        pl.BlockSpec((BLK, HID), lambda i: (i, 0)),
        _full((HID, HID // 2)),
        _full((1, HID // 2)),
        _full((HID // 2, HID // 4)),
        _full((1, HID // 4)),
        _full((HID // 4, D)),
        _full((1, D)),
        _full((HID, HID // 2)),
        _full((1, HID // 2)),
        _full((HID // 2, D)),
        _full((1, D)),
    ],
    out_specs=pl.BlockSpec((BLK,), lambda i: (i,)),
    out_shape=jax.ShapeDtypeStruct((B,), jnp.float32),
)


def kernel(user_feats, item_ids, item_feats_table, embed_table,
           uW1, ub1, uW2, ub2, uW3, ub3, iW1, ib1, iW2, ib2):
    item_field_ids = _fields_gather()(item_ids.reshape(B // CHUNK, CHUNK),
                                      item_feats_table)
    flat_ids = jnp.concatenate(
        [user_feats.reshape(-1), item_field_ids.reshape(-1)])
    ue_flat, ie_flat = _embed_gather()(flat_ids.reshape(N_IDS // CHUNK, CHUNK),
                                       embed_table)
    ue = ue_flat.reshape(B, HID)
    ie = ie_flat.reshape(B, HID)
    scores = _mlp_call(ue, ie,
                       uW1, ub1.reshape(1, -1), uW2, ub2.reshape(1, -1),
                       uW3, ub3.reshape(1, -1), iW1, ib1.reshape(1, -1),
                       iW2, ib2.reshape(1, -1))
    return scores
